# baseline (speedup 1.0000x reference)
import functools

import jax
import jax.numpy as jnp
from jax import lax
from jax.experimental import pallas as pl
from jax.experimental.pallas import tpu as pltpu
from jax.experimental.pallas import tpu_sc as plsc


def _proj_body(t_ref, w_ref, b_ref, o_ref):
    o_ref[...] = jnp.maximum(
        jnp.dot(t_ref[...], w_ref[...], preferred_element_type=jnp.float32)
        + b_ref[...],
        0.0,
    )


def _project(table, W, b):
    V, E = table.shape
    O = W.shape[1]
    BR = 1000
    return pl.pallas_call(
        _proj_body,
        grid=(V // BR,),
        in_specs=[
            pl.BlockSpec((BR, E), lambda i: (i, 0)),
            pl.BlockSpec((E, O), lambda i: (0, 0)),
            pl.BlockSpec((1, O), lambda i: (0, 0)),
        ],
        out_specs=pl.BlockSpec((BR, O), lambda i: (i, 0)),
        out_shape=jax.ShapeDtypeStruct((V, O), jnp.float32),
    )(table, W, b.reshape(1, O))


@functools.lru_cache(maxsize=None)
def _make_gather(V, O, Bm, S, SB, SP):
    info = plsc.get_sparse_core_info()
    NC, NS = info.num_cores, info.num_subcores
    NW = NC * NS
    assert Bm % NW == 0
    rpw = Bm // NW
    mesh = plsc.VectorSubcoreMesh(core_axis_name="c", subcore_axis_name="s")

    @functools.partial(
        pl.kernel,
        mesh=mesh,
        out_type=jax.ShapeDtypeStruct((Bm, S, O), jnp.float32),
        scratch_types=[
            pltpu.VMEM((rpw, SP), jnp.int32),
            pltpu.VMEM((2, SB, O), jnp.float32),
            pltpu.SemaphoreType.DMA,
            pltpu.SemaphoreType.DMA,
        ],
    )
    def gather(tbl_hbm, idx_hbm, out_hbm, idx_v, rows_v, sem0, sem1):
        wid = lax.axis_index("s") * NC + lax.axis_index("c")
        sems = (sem0, sem1)
        pltpu.sync_copy(idx_hbm.at[pl.ds(wid * rpw, rpw)], idx_v)
        row0 = wid * rpw

        def chunk_idx(r):
            return idx_v.at[r, pl.ds(0, SB)]

        for b in range(2):
            pltpu.async_copy(tbl_hbm.at[chunk_idx(b)], rows_v.at[b], sems[b])

        def step(i, carry):
            c = i * 2
            for b in range(2):
                r = c + b
                pltpu.make_async_copy(
                    tbl_hbm.at[chunk_idx(r)], rows_v.at[b], sems[b]
                ).wait()
                pltpu.sync_copy(
                    rows_v.at[b], out_hbm.at[row0 + r].at[pl.ds(0, SB)]
                )

                @pl.when(r + 2 < rpw)
                def _():
                    pltpu.async_copy(
                        tbl_hbm.at[chunk_idx(r + 2)], rows_v.at[b], sems[b]
                    )

            return carry

        lax.fori_loop(0, rpw // 2, step, 0)

    return gather


def _patch_body(idx_ref, a_ref, b_ref, big_ref, o_ref):
    del idx_ref, big_ref
    o_ref[0, 0, :] = a_ref[0, 0, :]
    o_ref[0, 1, :] = b_ref[0, 0, :]


@functools.lru_cache(maxsize=None)
def _make_patch(V, O, Bm, S, SB):
    grid_spec = pltpu.PrefetchScalarGridSpec(
        num_scalar_prefetch=1,
        grid=(Bm,),
        in_specs=[
            pl.BlockSpec((1, 1, O), lambda i, idx: (idx[2 * i], 0, 0)),
            pl.BlockSpec((1, 1, O), lambda i, idx: (idx[2 * i + 1], 0, 0)),
            pl.BlockSpec(memory_space=pl.ANY),
        ],
        out_specs=pl.BlockSpec((1, 8, O), lambda i, idx: (i, SB // 8, 0)),
    )
    return pl.pallas_call(
        _patch_body,
        grid_spec=grid_spec,
        out_shape=jax.ShapeDtypeStruct((Bm, S, O), jnp.float32),
        input_output_aliases={3: 0},
    )


def kernel(input_ids, table, W, b):
    Bm, S = input_ids.shape
    V, E = table.shape
    O = W.shape[1]
    SB = 48
    SP = 128
    ids = input_ids.astype(jnp.int32)
    proj = _project(table, W, b)
    ids_p = jnp.pad(ids, ((0, 0), (0, SP - S)))
    bulk = _make_gather(V, O, Bm, S, SB, SP)(proj, ids_p)
    tail = ids[:, SB:S].reshape(-1)
    proj3 = proj.reshape(V, 1, O)
    return _make_patch(V, O, Bm, S, SB)(tail, proj3, proj3, bulk)


# speedup vs baseline: 3.5704x; 3.5704x over previous
import functools

import jax
import jax.numpy as jnp
from jax import lax
from jax.experimental import pallas as pl
from jax.experimental.pallas import tpu as pltpu
from jax.experimental.pallas import tpu_sc as plsc


def _proj_body(t_ref, w_ref, b_ref, o_ref):
    o_ref[...] = jnp.maximum(
        jnp.dot(t_ref[...], w_ref[...], preferred_element_type=jnp.float32)
        + b_ref[...],
        0.0,
    )


def _project(table, W, b):
    V, E = table.shape
    O = W.shape[1]
    BR = 1000
    return pl.pallas_call(
        _proj_body,
        grid=(V // BR,),
        in_specs=[
            pl.BlockSpec((BR, E), lambda i: (i, 0)),
            pl.BlockSpec((E, O), lambda i: (0, 0)),
            pl.BlockSpec((1, O), lambda i: (0, 0)),
        ],
        out_specs=pl.BlockSpec((BR, O), lambda i: (i, 0)),
        out_shape=jax.ShapeDtypeStruct((V, O), jnp.float32),
    )(table, W, b.reshape(1, O))


@functools.lru_cache(maxsize=None)
def _make_gather(V, O, Bm, S, SB, SP, TCW):
    info = plsc.get_sparse_core_info()
    NC, NS = info.num_cores, info.num_subcores
    NW = NC * NS
    assert Bm % (2 * NW) == 0
    rpw = Bm // NW
    half = rpw // 2
    NT = S - SB
    tpw = rpw * NT
    assert tpw % TCW == 0
    tchunks = tpw // TCW
    mesh = plsc.VectorSubcoreMesh(core_axis_name="c", subcore_axis_name="s")

    @functools.partial(
        pl.kernel,
        mesh=mesh,
        out_type=(
            jax.ShapeDtypeStruct((Bm, S, O), jnp.float32),
            jax.ShapeDtypeStruct((Bm * NT, O), jnp.float32),
        ),
        scratch_types=[
            pltpu.VMEM((half, SP), jnp.int32),
            pltpu.VMEM((tpw,), jnp.int32),
            pltpu.VMEM((2, SB, O), jnp.float32),
            pltpu.VMEM((2, TCW, O), jnp.float32),
            pltpu.SemaphoreType.DMA,
            pltpu.SemaphoreType.DMA,
        ],
    )
    def gather(tbl_hbm, idx_hbm, tidx_hbm, out_hbm, tails_hbm,
               idx_v, tidx_v, rows_v, trows_v, sem0, sem1):
        wid = lax.axis_index("s") * NC + lax.axis_index("c")
        sems = (sem0, sem1)
        row0 = wid * rpw

        pltpu.sync_copy(tidx_hbm.at[pl.ds(wid * tpw, tpw)], tidx_v)
        trow0 = wid * tpw

        for b in range(2):
            pltpu.async_copy(
                tbl_hbm.at[tidx_v.at[pl.ds(b * TCW, TCW)]],
                trows_v.at[b], sems[b],
            )

        def tstep(i, carry):
            c = i * 2
            for b in range(2):
                r = c + b
                pltpu.make_async_copy(
                    tbl_hbm.at[tidx_v.at[pl.ds(r * TCW, TCW)]],
                    trows_v.at[b], sems[b],
                ).wait()
                pltpu.sync_copy(
                    trows_v.at[b], tails_hbm.at[pl.ds(trow0 + r * TCW, TCW)]
                )

                @pl.when(r + 2 < tchunks)
                def _():
                    pltpu.async_copy(
                        tbl_hbm.at[tidx_v.at[pl.ds((r + 2) * TCW, TCW)]],
                        trows_v.at[b], sems[b],
                    )

            return carry

        lax.fori_loop(0, tchunks // 2, tstep, 0)

        def run_half(h, carry):
            pltpu.sync_copy(
                idx_hbm.at[pl.ds(wid * rpw + h * half, half)], idx_v
            )
            base = row0 + h * half

            def chunk_idx(r):
                return idx_v.at[r, pl.ds(0, SB)]

            for b in range(2):
                pltpu.async_copy(
                    tbl_hbm.at[chunk_idx(b)], rows_v.at[b], sems[b]
                )

            def step(i, carry2):
                c = i * 2
                for b in range(2):
                    r = c + b
                    pltpu.make_async_copy(
                        tbl_hbm.at[chunk_idx(r)], rows_v.at[b], sems[b]
                    ).wait()
                    pltpu.sync_copy(
                        rows_v.at[b], out_hbm.at[base + r].at[pl.ds(0, SB)]
                    )

                    @pl.when(r + 2 < half)
                    def _():
                        pltpu.async_copy(
                            tbl_hbm.at[chunk_idx(r + 2)], rows_v.at[b], sems[b]
                        )

                return carry2

            lax.fori_loop(0, half // 2, step, 0)
            return carry

        lax.fori_loop(0, 2, run_half, 0)

    return gather


def _patch_body(t_ref, big_ref, o_ref):
    del big_ref
    n = o_ref.shape[0]
    o_ref[:, 0:2, :] = t_ref[...].reshape(n, 2, o_ref.shape[2])


@functools.lru_cache(maxsize=None)
def _make_patch(O, Bm, S, SB, BR):
    return pl.pallas_call(
        _patch_body,
        grid=(Bm // BR,),
        in_specs=[
            pl.BlockSpec((2 * BR, O), lambda i: (i, 0)),
            pl.BlockSpec(memory_space=pl.ANY),
        ],
        out_specs=pl.BlockSpec((BR, 8, O), lambda i: (i, SB // 8, 0)),
        out_shape=jax.ShapeDtypeStruct((Bm, S, O), jnp.float32),
        input_output_aliases={1: 0},
    )


def kernel(input_ids, table, W, b):
    Bm, S = input_ids.shape
    V, E = table.shape
    O = W.shape[1]
    SB = 48
    SP = 128
    TCW = 32
    BR = 64
    ids = input_ids.astype(jnp.int32)
    proj = _project(table, W, b)
    ids_p = jnp.pad(ids, ((0, 0), (0, SP - S)))
    tail_ids = ids[:, SB:S].reshape(-1)
    bulk, tails = _make_gather(V, O, Bm, S, SB, SP, TCW)(proj, ids_p, tail_ids)
    return _make_patch(O, Bm, S, SB, BR)(tails, bulk)


# speedup vs baseline: 8.2270x; 2.3042x over previous
import functools

import jax
import jax.numpy as jnp
from jax import lax
from jax.experimental import pallas as pl
from jax.experimental.pallas import tpu as pltpu
from jax.experimental.pallas import tpu_sc as plsc


def _proj_body(t_ref, w_ref, b_ref, o_ref):
    o_ref[...] = jnp.maximum(
        jnp.dot(t_ref[...], w_ref[...], preferred_element_type=jnp.float32)
        + b_ref[...],
        0.0,
    )


def _project(table, W, b):
    V, E = table.shape
    O = W.shape[1]
    BR = 1000
    return pl.pallas_call(
        _proj_body,
        grid=(V // BR,),
        in_specs=[
            pl.BlockSpec((BR, E), lambda i: (i, 0)),
            pl.BlockSpec((E, O), lambda i: (0, 0)),
            pl.BlockSpec((1, O), lambda i: (0, 0)),
        ],
        out_specs=pl.BlockSpec((BR, O), lambda i: (i, 0)),
        out_shape=jax.ShapeDtypeStruct((V, O), jnp.float32),
    )(table, W, b.reshape(1, O))


@functools.lru_cache(maxsize=None)
def _make_gather(V, O, Bm, S, CW):
    info = plsc.get_sparse_core_info()
    NC, NS = info.num_cores, info.num_subcores
    NW = NC * NS
    assert Bm % (NW * CW) == 0
    bpw = Bm // NW
    kpt = bpw // CW
    chunks = S * kpt
    mesh = plsc.VectorSubcoreMesh(core_axis_name="c", subcore_axis_name="s")

    @functools.partial(
        pl.kernel,
        mesh=mesh,
        out_type=jax.ShapeDtypeStruct((S, Bm, O), jnp.float32),
        scratch_types=[
            pltpu.VMEM((S, bpw), jnp.int32),
            pltpu.VMEM((2, CW, O), jnp.float32),
            pltpu.SemaphoreType.DMA,
            pltpu.SemaphoreType.DMA,
        ],
    )
    def gather(tbl_hbm, idx_hbm, out_hbm, idx_v, rows_v, sem0, sem1):
        wid = lax.axis_index("s") * NC + lax.axis_index("c")
        sems = (sem0, sem1)
        col0 = wid * bpw
        pltpu.sync_copy(idx_hbm.at[:, pl.ds(col0, bpw)], idx_v)

        def chunk_idx(c):
            t = c // kpt
            k = lax.rem(c, kpt)
            return idx_v.at[t, pl.ds(k * CW, CW)]

        for b in range(2):
            pltpu.async_copy(tbl_hbm.at[chunk_idx(b)], rows_v.at[b], sems[b])

        def step(i, carry):
            c0 = i * 2
            for b in range(2):
                c = c0 + b
                t = c // kpt
                k = lax.rem(c, kpt)
                pltpu.make_async_copy(
                    tbl_hbm.at[chunk_idx(c)], rows_v.at[b], sems[b]
                ).wait()
                pltpu.sync_copy(
                    rows_v.at[b],
                    out_hbm.at[t].at[pl.ds(col0 + k * CW, CW)],
                )

                @pl.when(c + 2 < chunks)
                def _():
                    pltpu.async_copy(
                        tbl_hbm.at[chunk_idx(c + 2)], rows_v.at[b], sems[b]
                    )

            return carry

        lax.fori_loop(0, chunks // 2, step, 0)

    return gather


def kernel(input_ids, table, W, b):
    Bm, S = input_ids.shape
    V, E = table.shape
    O = W.shape[1]
    CW = 64
    proj = _project(table, W, b)
    ids_t = jnp.transpose(input_ids.astype(jnp.int32))
    out_t = _make_gather(V, O, Bm, S, CW)(proj, ids_t)
    return jnp.transpose(out_t, (1, 0, 2))
